# baseline (speedup 1.0000x reference)
import functools

import jax
import jax.numpy as jnp
from jax import lax
from jax.experimental import pallas as pl
from jax.experimental.pallas import tpu as pltpu
from jax.experimental.pallas import tpu_sc as plsc

N_TOKENS = 8192
D_MODEL = 4096
N_PATHS = 16
NC = 2
NS = 16
NW = NC * NS
RPW = N_TOKENS // NW
CH = 8
NCHUNK = RPW // CH


def _dispatch_sc(x, pos2d):
    mesh = plsc.VectorSubcoreMesh(core_axis_name="c", subcore_axis_name="s")

    W = 16
    PR = RPW // 16

    @functools.partial(
        pl.kernel,
        out_type=jax.ShapeDtypeStruct((N_TOKENS, D_MODEL), jnp.float32),
        mesh=mesh,
        scratch_types=[
            pltpu.VMEM((PR, 16), jnp.int32),
            pltpu.SemaphoreType.DMA,
        ],
    )
    def dispatch(x_hbm, pos_hbm, out_hbm, pos_v, sem):
        wid = lax.axis_index("s") * NC + lax.axis_index("c")
        base = wid * RPW

        pltpu.sync_copy(pos_hbm.at[pl.ds(wid * PR, PR)], pos_v)

        copies = []
        for k in range(PR):
            v = pos_v[k]
            for j in range(16):
                r = k * 16 + j
                copies.append(pltpu.make_async_copy(
                    x_hbm.at[pl.ds(base + r, 1)],
                    out_hbm.at[pl.ds(v[j], 1)],
                    sem,
                ))
        for r in range(RPW):
            copies[r].start()
            if r >= W:
                copies[r - W].wait()
        for r in range(RPW - W, RPW):
            copies[r].wait()

    return dispatch(x, pos2d)


def kernel(inputs):
    n = inputs.shape[0]
    score = jax.random.normal(jax.random.key(42), (n, N_PATHS), dtype=jnp.float32)
    route = jnp.argmax(score, axis=1).astype(jnp.int32)

    onehot = (route[:, None] == jnp.arange(N_PATHS, dtype=jnp.int32)[None, :])
    prefix = jnp.cumsum(onehot.astype(jnp.int32), axis=0)
    counts = prefix[-1]
    rank = jnp.take_along_axis(prefix, route[:, None], axis=1)[:, 0] - 1
    ends = jnp.cumsum(counts)
    starts = ends - counts
    pos = starts[route] + rank

    route_sorted = jnp.sum(
        jnp.arange(n, dtype=jnp.int32)[:, None] >= ends[None, :], axis=1
    ).astype(jnp.int32)

    dispatched = _dispatch_sc(inputs, pos.reshape(n // 16, 16))
    return dispatched, route_sorted, counts


# speedup vs baseline: 29.0536x; 29.0536x over previous
import functools

import jax
import jax.numpy as jnp
from jax import lax
from jax.experimental import pallas as pl
from jax.experimental.pallas import tpu as pltpu
from jax.experimental.pallas import tpu_sc as plsc

N_TOKENS = 8192
D_MODEL = 4096
N_PATHS = 16
NC = 2
NS = 16
NW = NC * NS
RPW = N_TOKENS // NW
CH = 8
NCHUNK = RPW // CH


def _dispatch_sc(x, pos2d):
    mesh = plsc.VectorSubcoreMesh(core_axis_name="c", subcore_axis_name="s")

    NBUF = 3

    @functools.partial(
        pl.kernel,
        out_type=jax.ShapeDtypeStruct((N_TOKENS, D_MODEL), jnp.float32),
        mesh=mesh,
        scratch_types=[
            pltpu.VMEM((NCHUNK, CH), jnp.int32),
            pltpu.VMEM((CH, D_MODEL), jnp.float32),
            pltpu.VMEM((CH, D_MODEL), jnp.float32),
            pltpu.VMEM((CH, D_MODEL), jnp.float32),
            pltpu.SemaphoreType.DMA,
            pltpu.SemaphoreType.DMA,
            pltpu.SemaphoreType.DMA,
            pltpu.SemaphoreType.DMA,
            pltpu.SemaphoreType.DMA,
            pltpu.SemaphoreType.DMA,
        ],
    )
    def dispatch(x_hbm, pos_hbm, out_hbm, pos_v,
                 buf0, buf1, buf2, si0, si1, si2, so0, so1, so2):
        wid = lax.axis_index("s") * NC + lax.axis_index("c")
        base = wid * RPW
        buf = (buf0, buf1, buf2)
        sin = (si0, si1, si2)
        sout = (so0, so1, so2)

        pltpu.sync_copy(pos_hbm.at[pl.ds(wid * NCHUNK, NCHUNK)], pos_v)

        def start_in(k, b):
            pltpu.async_copy(x_hbm.at[pl.ds(base + k * CH, CH)], buf[b], sin[b])

        def wait_in(k, b):
            pltpu.make_async_copy(
                x_hbm.at[pl.ds(base + k * CH, CH)], buf[b], sin[b]).wait()

        def start_out(k, b):
            pltpu.async_copy(buf[b], out_hbm.at[pos_v.at[k]], sout[b])

        def wait_out(k, b):
            pltpu.make_async_copy(
                buf[b], out_hbm.at[pos_v.at[k]], sout[b]).wait()

        for k in range(NBUF - 1):
            start_in(k, k % NBUF)
        for k in range(NCHUNK):
            b = k % NBUF
            p = k + NBUF - 1
            if p < NCHUNK:
                bp = p % NBUF
                if k >= 1:
                    wait_out(k - 1, bp)
                start_in(p, bp)
            wait_in(k, b)
            start_out(k, b)
        for k in range(NCHUNK - NBUF, NCHUNK):
            if k >= 0:
                wait_out(k, k % NBUF)

    return dispatch(x, pos2d)


def kernel(inputs):
    n = inputs.shape[0]
    score = jax.random.normal(jax.random.key(42), (n, N_PATHS), dtype=jnp.float32)
    route = jnp.argmax(score, axis=1).astype(jnp.int32)

    onehot = (route[:, None] == jnp.arange(N_PATHS, dtype=jnp.int32)[None, :])
    prefix = jnp.cumsum(onehot.astype(jnp.int32), axis=0)
    counts = prefix[-1]
    rank = jnp.take_along_axis(prefix, route[:, None], axis=1)[:, 0] - 1
    ends = jnp.cumsum(counts)
    starts = ends - counts
    pos = starts[route] + rank

    route_sorted = jnp.sum(
        jnp.arange(n, dtype=jnp.int32)[:, None] >= ends[None, :], axis=1
    ).astype(jnp.int32)

    dispatched = _dispatch_sc(inputs, pos.reshape(n // CH, CH))
    return dispatched, route_sorted, counts
